# baseline (speedup 1.0000x reference)
import functools

import jax
import jax.numpy as jnp
from jax import lax
from jax.experimental import pallas as pl
from jax.experimental.pallas import tpu as pltpu
from jax.experimental.pallas import tpu_sc as plsc

K = 8
EPS = 1e-5
TH = 7
LANES = 16


def _stage_a_body(ct_ref, xt_ref, g_ref, b_ref,
                  mc_ref, s1t_ref, s2t_ref, m2_ref, c2_ref, gt_ref, bt_ref,
                  s1acc, s2acc):
    i = pl.program_id(0)
    n = pl.num_programs(0)
    P = TH * 14
    B, C = s1acc.shape
    ctb = ct_ref[...].reshape(P, K, C)
    xb = xt_ref[...].reshape(P, B, C)
    pmc = lax.dot_general(ctb, xb, (((2,), (2,)), ((0,), (0,))),
                          preferred_element_type=jnp.float32)
    pmc = jnp.sum(pmc, axis=0)
    ps1 = jnp.sum(xb, axis=0)
    ps2 = jnp.sum(xb * xb, axis=0)

    pc2 = jnp.sum(jnp.sum(ctb * ctb, axis=0), axis=1, keepdims=True)

    @pl.when(i == 0)
    def _():
        mc_ref[...] = pmc
        s1acc[...] = ps1
        s2acc[...] = ps2
        c2_ref[...] = pc2

    @pl.when(i > 0)
    def _():
        mc_ref[...] += pmc
        s1acc[...] += ps1
        s2acc[...] += ps2
        c2_ref[...] += pc2

    @pl.when(i == n - 1)
    def _():
        s2f = s2acc[...]
        s1t_ref[...] = s1acc[...].T
        s2t_ref[...] = s2f.T
        m2_ref[...] = jnp.sum(s2f, axis=1, keepdims=True)
        gt_ref[...] = g_ref[...].T
        bt_ref[...] = b_ref[...].T


def _stage_c_body(xt_ref, sc8_ref, sh8_ref, oh_ref, o_ref):
    xb = xt_ref[...]
    oh = oh_ref[...]
    dn = (((1,), (1,)), ((), ()))
    sb = lax.dot_general(oh, sc8_ref[...], dn,
                         preferred_element_type=jnp.float32)
    tb = lax.dot_general(oh, sh8_ref[...], dn,
                         preferred_element_type=jnp.float32)
    o_ref[...] = xb * sb[None, None, :, :] + tb[None, None, :, :]


def _fast_rsqrt(v):
    bits = plsc.bitcast(v, jnp.int32)
    y = plsc.bitcast(jnp.int32(0x5F3759DF) - lax.shift_right_logical(bits, 1),
                     jnp.float32)
    for _ in range(3):
        y = y * (1.5 - 0.5 * v * y * y)
    return y


def _sc_middle(mc_t, m2, c2, s1t, s2t, gt, bt, hw):
    C, B = s1t.shape
    nsub = C // LANES
    nb2 = B // 2
    f32 = jnp.float32
    mesh = plsc.VectorSubcoreMesh(core_axis_name="c", subcore_axis_name="s")

    @functools.partial(
        pl.kernel, mesh=mesh,
        compiler_params=pltpu.CompilerParams(needs_layout_passes=False),
        out_type=[jax.ShapeDtypeStruct((C, K), f32),
                  jax.ShapeDtypeStruct((C, K), f32),
                  jax.ShapeDtypeStruct((B, K), f32)],
        scratch_types=[
            pltpu.VMEM((K, B), f32),
            pltpu.VMEM((B, 1), f32),
            pltpu.VMEM((K, 1), f32),
            pltpu.VMEM((LANES, B), f32),
            pltpu.VMEM((LANES, B), f32),
            pltpu.VMEM((LANES, K), f32),
            pltpu.VMEM((LANES, K), f32),
            pltpu.VMEM((K, LANES), f32),
            pltpu.VMEM((K, LANES), f32),
            pltpu.VMEM((K, LANES), f32),
            pltpu.VMEM((LANES, K), f32),
            pltpu.VMEM((LANES, K), f32),
            pltpu.VMEM((B, K), f32),
        ],
    )
    def middle(mc_ref, m2_ref, c2_ref, s1_ref, s2_ref, g_ref, b_ref,
               sc8_ref, sh8_ref, oh_ref,
               mc_v, m2_v, c2_v, s1_v, s2_v, g_v, b_v,
               acc1, acc2, cntv, oc_v, os_v, oh_v):
        wid = lax.axis_index("s") * 2 + lax.axis_index("c")

        @pl.when(wid < nsub)
        def _():
            c0 = wid * LANES
            pltpu.sync_copy(mc_ref, mc_v)
            pltpu.sync_copy(m2_ref, m2_v)
            pltpu.sync_copy(c2_ref, c2_v)
            pltpu.sync_copy(s1_ref.at[pl.ds(c0, LANES), :], s1_v)
            pltpu.sync_copy(s2_ref.at[pl.ds(c0, LANES), :], s2_v)
            pltpu.sync_copy(g_ref.at[pl.ds(c0, LANES), :], g_v)
            pltpu.sync_copy(b_ref.at[pl.ds(c0, LANES), :], b_v)

            iota = lax.iota(jnp.int32, LANES)
            zeros_i = jnp.zeros((LANES,), jnp.int32)
            rowsel = jnp.bitwise_and(iota, 7)
            half_lo = iota < 8
            big = f32(3.0e38)
            c2v = plsc.load_gather(c2_v, [rowsel, zeros_i])

            for k in range(K):
                acc1[k] = jnp.zeros((LANES,), f32)
                acc2[k] = jnp.zeros((LANES,), f32)
                cntv[k] = jnp.zeros((LANES,), f32)

            def _assign(r):
                b0 = 2 * r
                b1 = b0 + 1
                colidx = jnp.where(half_lo, b0, b1)
                rows = plsc.load_gather(mc_v, [rowsel, colidx])
                m2v = plsc.load_gather(m2_v, [colidx, zeros_i])
                diff = jnp.abs(m2v + c2v - 2.0 * rows)
                dA = jnp.where(half_lo, diff, big)
                kA = plsc.all_reduce_ffs(dA == jnp.min(dA))
                dB = jnp.where(half_lo, big, diff)
                kB = plsc.all_reduce_ffs(dB == jnp.min(dB)) - 8
                return kA, kB

            one = jnp.ones((LANES,), f32)
            zero = jnp.zeros((LANES,), f32)
            oh_mask = (zeros_i + wid) == 0

            for r in range(nb2):
                b0 = 2 * r
                b1 = b0 + 1
                kA, kB = _assign(r)
                i0 = zeros_i + b0
                i1 = zeros_i + b1
                r1a = plsc.load_gather(s1_v, [iota, i0])
                r1b = plsc.load_gather(s1_v, [iota, i1])
                r2a = plsc.load_gather(s2_v, [iota, i0])
                r2b = plsc.load_gather(s2_v, [iota, i1])
                for k in range(K):
                    wa = jnp.where(kA == k, one, zero)
                    wb = jnp.where(kB == k, one, zero)
                    acc1[k] = acc1[k] + r1a * wa + r1b * wb
                    acc2[k] = acc2[k] + r2a * wa + r2b * wb
                    cntv[k] = cntv[k] + wa + wb
                rowi = jnp.where(half_lo, b0, b1)
                kpair = jnp.where(half_lo, kA + zeros_i, kB + zeros_i)
                ohval = jnp.where(rowsel == kpair, one, zero)
                plsc.store_scatter(oh_v, [rowi, rowsel], ohval, mask=oh_mask)

            for k in range(K):
                denom = jnp.maximum(cntv[k] * hw, 1.0)
                mean = acc1[k] / denom
                var = acc2[k] / denom - mean * mean
                inv = _fast_rsqrt(var + EPS)
                gk = plsc.load_gather(g_v, [iota, zeros_i + k])
                bk = plsc.load_gather(b_v, [iota, zeros_i + k])
                sck = gk * inv
                shk = bk - mean * sck
                ki = zeros_i + k
                plsc.store_scatter(oc_v, [iota, ki], sck)
                plsc.store_scatter(os_v, [iota, ki], shk)

            pltpu.sync_copy(oc_v, sc8_ref.at[pl.ds(c0, LANES), :])
            pltpu.sync_copy(os_v, sh8_ref.at[pl.ds(c0, LANES), :])

            @pl.when(wid == 0)
            def _():
                pltpu.sync_copy(oh_v, oh_ref)

    return middle(mc_t, m2, c2, s1t, s2t, gt, bt)


def kernel(x, c, gamma, beta):
    B, C, H, W = x.shape
    HW = H * W
    xt = jnp.transpose(x, (2, 3, 0, 1))
    ct = jnp.transpose(c.reshape(K, C, H, W), (2, 3, 0, 1))
    grid = H // TH
    f32 = jnp.float32

    mc_t, s1t, s2t, m2, c2, gt, bt = pl.pallas_call(
        _stage_a_body,
        grid=(grid,),
        in_specs=[
            pl.BlockSpec((TH, W, K, C), lambda i: (i, 0, 0, 0)),
            pl.BlockSpec((TH, W, B, C), lambda i: (i, 0, 0, 0)),
            pl.BlockSpec((K, C), lambda i: (0, 0)),
            pl.BlockSpec((K, C), lambda i: (0, 0)),
        ],
        out_specs=[
            pl.BlockSpec((K, B), lambda i: (0, 0)),
            pl.BlockSpec((C, B), lambda i: (0, 0)),
            pl.BlockSpec((C, B), lambda i: (0, 0)),
            pl.BlockSpec((B, 1), lambda i: (0, 0)),
            pl.BlockSpec((K, 1), lambda i: (0, 0)),
            pl.BlockSpec((C, K), lambda i: (0, 0)),
            pl.BlockSpec((C, K), lambda i: (0, 0)),
        ],
        out_shape=[
            jax.ShapeDtypeStruct((K, B), f32),
            jax.ShapeDtypeStruct((C, B), f32),
            jax.ShapeDtypeStruct((C, B), f32),
            jax.ShapeDtypeStruct((B, 1), f32),
            jax.ShapeDtypeStruct((K, 1), f32),
            jax.ShapeDtypeStruct((C, K), f32),
            jax.ShapeDtypeStruct((C, K), f32),
        ],
        scratch_shapes=[
            pltpu.VMEM((B, C), f32),
            pltpu.VMEM((B, C), f32),
        ],
    )(ct, xt, gamma, beta)

    sc8, sh8, oh = _sc_middle(mc_t, m2, c2, s1t, s2t, gt, bt, float(HW))

    outT = pl.pallas_call(
        _stage_c_body,
        grid=(grid,),
        in_specs=[
            pl.BlockSpec((TH, W, B, C), lambda i: (i, 0, 0, 0)),
            pl.BlockSpec((C, K), lambda i: (0, 0)),
            pl.BlockSpec((C, K), lambda i: (0, 0)),
            pl.BlockSpec((B, K), lambda i: (0, 0)),
        ],
        out_specs=pl.BlockSpec((TH, W, B, C), lambda i: (i, 0, 0, 0)),
        out_shape=jax.ShapeDtypeStruct((H, W, B, C), f32),
    )(xt, sc8, sh8, oh)

    return jnp.transpose(outT, (2, 3, 0, 1))


# speedup vs baseline: 1.0365x; 1.0365x over previous
import functools

import jax
import jax.numpy as jnp
from jax import lax
from jax.experimental import pallas as pl
from jax.experimental.pallas import tpu as pltpu
from jax.experimental.pallas import tpu_sc as plsc

K = 8
EPS = 1e-5
TH = 7
LANES = 16


def _stage_a_body(ct_ref, xt_ref, g_ref, b_ref,
                  mc_ref, s1t_ref, s2t_ref, m2_ref, c2_ref, gt_ref, bt_ref,
                  s1acc, s2acc):
    i = pl.program_id(0)
    n = pl.num_programs(0)
    P = TH * 14
    B, C = s1acc.shape
    ctb = ct_ref[...].reshape(P, K, C)
    xb = xt_ref[...].reshape(P, B, C)
    pmc = lax.dot_general(ctb, xb, (((2,), (2,)), ((0,), (0,))),
                          preferred_element_type=jnp.float32)
    pmc = jnp.sum(pmc, axis=0)
    ps1 = jnp.sum(xb, axis=0)
    ps2 = jnp.sum(xb * xb, axis=0)

    pc2 = jnp.sum(jnp.sum(ctb * ctb, axis=0), axis=1, keepdims=True)

    @pl.when(i == 0)
    def _():
        mc_ref[...] = pmc
        s1acc[...] = ps1
        s2acc[...] = ps2
        c2_ref[...] = pc2

    @pl.when(i > 0)
    def _():
        mc_ref[...] += pmc
        s1acc[...] += ps1
        s2acc[...] += ps2
        c2_ref[...] += pc2

    @pl.when(i == n - 1)
    def _():
        s2f = s2acc[...]
        s1t_ref[...] = s1acc[...].T
        s2t_ref[...] = s2f.T
        m2_ref[...] = jnp.sum(s2f, axis=1, keepdims=True)
        gt_ref[...] = g_ref[...].T
        bt_ref[...] = b_ref[...].T


def _stage_c_body(xt_ref, sc8_ref, sh8_ref, oh_ref, o_ref):
    xb = xt_ref[...]
    oh = oh_ref[...]
    dn = (((1,), (1,)), ((), ()))
    sb = lax.dot_general(oh, sc8_ref[...], dn,
                         preferred_element_type=jnp.float32)
    tb = lax.dot_general(oh, sh8_ref[...], dn,
                         preferred_element_type=jnp.float32)
    o_ref[...] = xb * sb[None, None, :, :] + tb[None, None, :, :]


def _fast_rsqrt(v):
    bits = plsc.bitcast(v, jnp.int32)
    y = plsc.bitcast(jnp.int32(0x5F3759DF) - lax.shift_right_logical(bits, 1),
                     jnp.float32)
    for _ in range(3):
        y = y * (1.5 - 0.5 * v * y * y)
    return y


def _sc_middle(mc_t, m2, c2, s1t, s2t, gt, bt, hw):
    C, B = s1t.shape
    nsub = C // LANES
    nb2 = B // 2
    f32 = jnp.float32
    mesh = plsc.VectorSubcoreMesh(core_axis_name="c", subcore_axis_name="s")

    @functools.partial(
        pl.kernel, mesh=mesh,
        compiler_params=pltpu.CompilerParams(needs_layout_passes=False),
        out_type=[jax.ShapeDtypeStruct((C, K), f32),
                  jax.ShapeDtypeStruct((C, K), f32),
                  jax.ShapeDtypeStruct((B, K), f32)],
        scratch_types=[
            pltpu.VMEM((K, B), f32),
            pltpu.VMEM((B, 1), f32),
            pltpu.VMEM((K, 1), f32),
            pltpu.VMEM((LANES, B), f32),
            pltpu.VMEM((LANES, B), f32),
            pltpu.VMEM((LANES, K), f32),
            pltpu.VMEM((LANES, K), f32),
            pltpu.VMEM((K, LANES), f32),
            pltpu.VMEM((K, LANES), f32),
            pltpu.VMEM((K, LANES), f32),
            pltpu.VMEM((LANES, K), f32),
            pltpu.VMEM((LANES, K), f32),
            pltpu.VMEM((B, K), f32),
        ],
    )
    def middle(mc_ref, m2_ref, c2_ref, s1_ref, s2_ref, g_ref, b_ref,
               sc8_ref, sh8_ref, oh_ref,
               mc_v, m2_v, c2_v, s1_v, s2_v, g_v, b_v,
               acc1, acc2, cntv, oc_v, os_v, oh_v):
        wid = lax.axis_index("s") * 2 + lax.axis_index("c")

        @pl.when(wid < nsub)
        def _():
            c0 = wid * LANES
            pltpu.sync_copy(mc_ref, mc_v)
            pltpu.sync_copy(m2_ref, m2_v)
            pltpu.sync_copy(c2_ref, c2_v)
            pltpu.sync_copy(s1_ref.at[pl.ds(c0, LANES), :], s1_v)
            pltpu.sync_copy(s2_ref.at[pl.ds(c0, LANES), :], s2_v)
            pltpu.sync_copy(g_ref.at[pl.ds(c0, LANES), :], g_v)
            pltpu.sync_copy(b_ref.at[pl.ds(c0, LANES), :], b_v)

            iota = lax.iota(jnp.int32, LANES)
            zeros_i = jnp.zeros((LANES,), jnp.int32)
            rowsel = jnp.bitwise_and(iota, 7)
            half_lo = iota < 8
            big = f32(3.0e38)
            c2v = plsc.load_gather(c2_v, [rowsel, zeros_i])

            for k in range(K):
                acc1[k] = jnp.zeros((LANES,), f32)
                acc2[k] = jnp.zeros((LANES,), f32)
                cntv[k] = jnp.zeros((LANES,), f32)

            def _assign(r):
                b0 = 2 * r
                b1 = b0 + 1
                colidx = jnp.where(half_lo, b0, b1)
                rows = plsc.load_gather(mc_v, [rowsel, colidx])
                m2v = plsc.load_gather(m2_v, [colidx, zeros_i])
                diff = jnp.abs(m2v + c2v - 2.0 * rows)
                dA = jnp.where(half_lo, diff, big)
                kA = plsc.all_reduce_ffs(dA == jnp.min(dA))
                dB = jnp.where(half_lo, big, diff)
                kB = plsc.all_reduce_ffs(dB == jnp.min(dB)) - 8
                return kA, kB

            one = jnp.ones((LANES,), f32)
            zero = jnp.zeros((LANES,), f32)
            oh_mask = (zeros_i + wid) == 0

            def loop1(r, carry):
                b0 = 2 * r
                b1 = b0 + 1
                kA, kB = _assign(r)
                i0 = zeros_i + b0
                i1 = zeros_i + b1
                r1a = plsc.load_gather(s1_v, [iota, i0])
                r1b = plsc.load_gather(s1_v, [iota, i1])
                r2a = plsc.load_gather(s2_v, [iota, i0])
                r2b = plsc.load_gather(s2_v, [iota, i1])
                for k in range(K):
                    wa = jnp.where(kA == k, one, zero)
                    wb = jnp.where(kB == k, one, zero)
                    acc1[k] = acc1[k] + r1a * wa + r1b * wb
                    acc2[k] = acc2[k] + r2a * wa + r2b * wb
                    cntv[k] = cntv[k] + wa + wb
                rowi = jnp.where(half_lo, b0, b1)
                kpair = jnp.where(half_lo, kA + zeros_i, kB + zeros_i)
                ohval = jnp.where(rowsel == kpair, one, zero)
                plsc.store_scatter(oh_v, [rowi, rowsel], ohval, mask=oh_mask)
                return carry

            lax.fori_loop(0, nb2, loop1, 0)

            for k in range(K):
                denom = jnp.maximum(cntv[k] * hw, 1.0)
                mean = acc1[k] / denom
                var = acc2[k] / denom - mean * mean
                inv = _fast_rsqrt(var + EPS)
                gk = plsc.load_gather(g_v, [iota, zeros_i + k])
                bk = plsc.load_gather(b_v, [iota, zeros_i + k])
                sck = gk * inv
                shk = bk - mean * sck
                ki = zeros_i + k
                plsc.store_scatter(oc_v, [iota, ki], sck)
                plsc.store_scatter(os_v, [iota, ki], shk)

            pltpu.sync_copy(oc_v, sc8_ref.at[pl.ds(c0, LANES), :])
            pltpu.sync_copy(os_v, sh8_ref.at[pl.ds(c0, LANES), :])

            @pl.when(wid == 0)
            def _():
                pltpu.sync_copy(oh_v, oh_ref)

    return middle(mc_t, m2, c2, s1t, s2t, gt, bt)


def kernel(x, c, gamma, beta):
    B, C, H, W = x.shape
    HW = H * W
    xt = jnp.transpose(x, (2, 3, 0, 1))
    ct = jnp.transpose(c.reshape(K, C, H, W), (2, 3, 0, 1))
    grid = H // TH
    f32 = jnp.float32

    mc_t, s1t, s2t, m2, c2, gt, bt = pl.pallas_call(
        _stage_a_body,
        grid=(grid,),
        in_specs=[
            pl.BlockSpec((TH, W, K, C), lambda i: (i, 0, 0, 0)),
            pl.BlockSpec((TH, W, B, C), lambda i: (i, 0, 0, 0)),
            pl.BlockSpec((K, C), lambda i: (0, 0)),
            pl.BlockSpec((K, C), lambda i: (0, 0)),
        ],
        out_specs=[
            pl.BlockSpec((K, B), lambda i: (0, 0)),
            pl.BlockSpec((C, B), lambda i: (0, 0)),
            pl.BlockSpec((C, B), lambda i: (0, 0)),
            pl.BlockSpec((B, 1), lambda i: (0, 0)),
            pl.BlockSpec((K, 1), lambda i: (0, 0)),
            pl.BlockSpec((C, K), lambda i: (0, 0)),
            pl.BlockSpec((C, K), lambda i: (0, 0)),
        ],
        out_shape=[
            jax.ShapeDtypeStruct((K, B), f32),
            jax.ShapeDtypeStruct((C, B), f32),
            jax.ShapeDtypeStruct((C, B), f32),
            jax.ShapeDtypeStruct((B, 1), f32),
            jax.ShapeDtypeStruct((K, 1), f32),
            jax.ShapeDtypeStruct((C, K), f32),
            jax.ShapeDtypeStruct((C, K), f32),
        ],
        scratch_shapes=[
            pltpu.VMEM((B, C), f32),
            pltpu.VMEM((B, C), f32),
        ],
    )(ct, xt, gamma, beta)

    sc8, sh8, oh = _sc_middle(mc_t, m2, c2, s1t, s2t, gt, bt, float(HW))

    outT = pl.pallas_call(
        _stage_c_body,
        grid=(grid,),
        in_specs=[
            pl.BlockSpec((TH, W, B, C), lambda i: (i, 0, 0, 0)),
            pl.BlockSpec((C, K), lambda i: (0, 0)),
            pl.BlockSpec((C, K), lambda i: (0, 0)),
            pl.BlockSpec((B, K), lambda i: (0, 0)),
        ],
        out_specs=pl.BlockSpec((TH, W, B, C), lambda i: (i, 0, 0, 0)),
        out_shape=jax.ShapeDtypeStruct((H, W, B, C), f32),
    )(xt, sc8, sh8, oh)

    return jnp.transpose(outT, (2, 3, 0, 1))


# speedup vs baseline: 1.5272x; 1.4734x over previous
import functools

import jax
import jax.numpy as jnp
from jax import lax
from jax.experimental import pallas as pl
from jax.experimental.pallas import tpu as pltpu
from jax.experimental.pallas import tpu_sc as plsc

K = 8
EPS = 1e-5
TH = 7
LANES = 16


def _stage_a_body(ct_ref, xt_ref, g_ref, b_ref,
                  mc_ref, s1t_ref, s2t_ref, m2_ref, c2_ref, gt_ref, bt_ref,
                  s1acc, s2acc):
    i = pl.program_id(0)
    n = pl.num_programs(0)
    P = TH * 14
    B, C = s1acc.shape
    ctb = ct_ref[...]
    xb = xt_ref[...].reshape(P, B, C)
    pmc = lax.dot_general(ctb, xb, (((2,), (2,)), ((0,), (0,))),
                          preferred_element_type=jnp.float32)
    pmc = jnp.sum(pmc, axis=0)
    ps1 = jnp.sum(xb, axis=0)
    ps2 = jnp.sum(xb * xb, axis=0)

    pc2 = jnp.sum(jnp.sum(ctb * ctb, axis=0), axis=1, keepdims=True)

    @pl.when(i == 0)
    def _():
        mc_ref[...] = pmc
        s1acc[...] = ps1
        s2acc[...] = ps2
        c2_ref[...] = pc2

    @pl.when(i > 0)
    def _():
        mc_ref[...] += pmc
        s1acc[...] += ps1
        s2acc[...] += ps2
        c2_ref[...] += pc2

    @pl.when(i == n - 1)
    def _():
        s2f = s2acc[...]
        s1t_ref[...] = s1acc[...].T
        s2t_ref[...] = s2f.T
        m2_ref[...] = jnp.sum(s2f, axis=1, keepdims=True)
        gt_ref[...] = g_ref[...].T
        bt_ref[...] = b_ref[...].T


def _stage_c_body(xt_ref, sc8_ref, sh8_ref, oh_ref, o_ref):
    xb = xt_ref[...]
    oh = oh_ref[...]
    dn = (((1,), (1,)), ((), ()))
    sb = lax.dot_general(oh, sc8_ref[...], dn,
                         preferred_element_type=jnp.float32)
    tb = lax.dot_general(oh, sh8_ref[...], dn,
                         preferred_element_type=jnp.float32)
    o_ref[...] = xb * sb[None, None, :, :] + tb[None, None, :, :]


def _fast_rsqrt(v):
    bits = plsc.bitcast(v, jnp.int32)
    y = plsc.bitcast(jnp.int32(0x5F3759DF) - lax.shift_right_logical(bits, 1),
                     jnp.float32)
    for _ in range(3):
        y = y * (1.5 - 0.5 * v * y * y)
    return y


def _sc_middle(mc_t, m2, c2, s1t, s2t, gt, bt, hw):
    C, B = s1t.shape
    nsub = C // LANES
    nb2 = B // 2
    f32 = jnp.float32
    mesh = plsc.VectorSubcoreMesh(core_axis_name="c", subcore_axis_name="s")

    @functools.partial(
        pl.kernel, mesh=mesh,
        compiler_params=pltpu.CompilerParams(needs_layout_passes=False),
        out_type=[jax.ShapeDtypeStruct((C, K), f32),
                  jax.ShapeDtypeStruct((C, K), f32),
                  jax.ShapeDtypeStruct((B, K), f32)],
        scratch_types=[
            pltpu.VMEM((K, B), f32),
            pltpu.VMEM((B, 1), f32),
            pltpu.VMEM((K, 1), f32),
            pltpu.VMEM((LANES, B), f32),
            pltpu.VMEM((LANES, B), f32),
            pltpu.VMEM((LANES, K), f32),
            pltpu.VMEM((LANES, K), f32),
            pltpu.VMEM((K, LANES), f32),
            pltpu.VMEM((K, LANES), f32),
            pltpu.VMEM((K, LANES), f32),
            pltpu.VMEM((LANES, K), f32),
            pltpu.VMEM((LANES, K), f32),
            pltpu.VMEM((B, K), f32),
        ],
    )
    def middle(mc_ref, m2_ref, c2_ref, s1_ref, s2_ref, g_ref, b_ref,
               sc8_ref, sh8_ref, oh_ref,
               mc_v, m2_v, c2_v, s1_v, s2_v, g_v, b_v,
               acc1, acc2, cntv, oc_v, os_v, oh_v):
        wid = lax.axis_index("s") * 2 + lax.axis_index("c")

        @pl.when(wid < nsub)
        def _():
            c0 = wid * LANES
            pltpu.sync_copy(mc_ref, mc_v)
            pltpu.sync_copy(m2_ref, m2_v)
            pltpu.sync_copy(c2_ref, c2_v)
            pltpu.sync_copy(s1_ref.at[pl.ds(c0, LANES), :], s1_v)
            pltpu.sync_copy(s2_ref.at[pl.ds(c0, LANES), :], s2_v)
            pltpu.sync_copy(g_ref.at[pl.ds(c0, LANES), :], g_v)
            pltpu.sync_copy(b_ref.at[pl.ds(c0, LANES), :], b_v)

            iota = lax.iota(jnp.int32, LANES)
            zeros_i = jnp.zeros((LANES,), jnp.int32)
            rowsel = jnp.bitwise_and(iota, 7)
            half_lo = iota < 8
            big = f32(3.0e38)
            c2v = plsc.load_gather(c2_v, [rowsel, zeros_i])

            for k in range(K):
                acc1[k] = jnp.zeros((LANES,), f32)
                acc2[k] = jnp.zeros((LANES,), f32)
                cntv[k] = jnp.zeros((LANES,), f32)

            def _assign(r):
                b0 = 2 * r
                b1 = b0 + 1
                colidx = jnp.where(half_lo, b0, b1)
                rows = plsc.load_gather(mc_v, [rowsel, colidx])
                m2v = plsc.load_gather(m2_v, [colidx, zeros_i])
                diff = jnp.abs(m2v + c2v - 2.0 * rows)
                dA = jnp.where(half_lo, diff, big)
                kA = plsc.all_reduce_ffs(dA == jnp.min(dA))
                dB = jnp.where(half_lo, big, diff)
                kB = plsc.all_reduce_ffs(dB == jnp.min(dB)) - 8
                return kA, kB

            one = jnp.ones((LANES,), f32)
            zero = jnp.zeros((LANES,), f32)
            oh_mask = (zeros_i + wid) == 0

            def loop1(r, carry):
                b0 = 2 * r
                b1 = b0 + 1
                kA, kB = _assign(r)
                i0 = zeros_i + b0
                i1 = zeros_i + b1
                r1a = plsc.load_gather(s1_v, [iota, i0])
                r1b = plsc.load_gather(s1_v, [iota, i1])
                r2a = plsc.load_gather(s2_v, [iota, i0])
                r2b = plsc.load_gather(s2_v, [iota, i1])
                for k in range(K):
                    wa = jnp.where(kA == k, one, zero)
                    wb = jnp.where(kB == k, one, zero)
                    acc1[k] = acc1[k] + r1a * wa + r1b * wb
                    acc2[k] = acc2[k] + r2a * wa + r2b * wb
                    cntv[k] = cntv[k] + wa + wb
                rowi = jnp.where(half_lo, b0, b1)
                kpair = jnp.where(half_lo, kA + zeros_i, kB + zeros_i)
                ohval = jnp.where(rowsel == kpair, one, zero)
                plsc.store_scatter(oh_v, [rowi, rowsel], ohval, mask=oh_mask)
                return carry

            lax.fori_loop(0, nb2, loop1, 0)

            for k in range(K):
                denom = jnp.maximum(cntv[k] * hw, 1.0)
                mean = acc1[k] / denom
                var = acc2[k] / denom - mean * mean
                inv = _fast_rsqrt(var + EPS)
                gk = plsc.load_gather(g_v, [iota, zeros_i + k])
                bk = plsc.load_gather(b_v, [iota, zeros_i + k])
                sck = gk * inv
                shk = bk - mean * sck
                ki = zeros_i + k
                plsc.store_scatter(oc_v, [iota, ki], sck)
                plsc.store_scatter(os_v, [iota, ki], shk)

            pltpu.sync_copy(oc_v, sc8_ref.at[pl.ds(c0, LANES), :])
            pltpu.sync_copy(os_v, sh8_ref.at[pl.ds(c0, LANES), :])

            @pl.when(wid == 0)
            def _():
                pltpu.sync_copy(oh_v, oh_ref)

    return middle(mc_t, m2, c2, s1t, s2t, gt, bt)


def kernel(x, c, gamma, beta):
    B, C, H, W = x.shape
    HW = H * W
    xt = jnp.transpose(x, (2, 3, 0, 1))
    ct = jnp.transpose(c.reshape(K, C, HW), (2, 0, 1))
    grid = H // TH
    f32 = jnp.float32

    mc_t, s1t, s2t, m2, c2, gt, bt = pl.pallas_call(
        _stage_a_body,
        grid=(grid,),
        in_specs=[
            pl.BlockSpec((TH * W, K, C), lambda i: (i, 0, 0)),
            pl.BlockSpec((TH, W, B, C), lambda i: (i, 0, 0, 0)),
            pl.BlockSpec((K, C), lambda i: (0, 0)),
            pl.BlockSpec((K, C), lambda i: (0, 0)),
        ],
        out_specs=[
            pl.BlockSpec((K, B), lambda i: (0, 0)),
            pl.BlockSpec((C, B), lambda i: (0, 0)),
            pl.BlockSpec((C, B), lambda i: (0, 0)),
            pl.BlockSpec((B, 1), lambda i: (0, 0)),
            pl.BlockSpec((K, 1), lambda i: (0, 0)),
            pl.BlockSpec((C, K), lambda i: (0, 0)),
            pl.BlockSpec((C, K), lambda i: (0, 0)),
        ],
        out_shape=[
            jax.ShapeDtypeStruct((K, B), f32),
            jax.ShapeDtypeStruct((C, B), f32),
            jax.ShapeDtypeStruct((C, B), f32),
            jax.ShapeDtypeStruct((B, 1), f32),
            jax.ShapeDtypeStruct((K, 1), f32),
            jax.ShapeDtypeStruct((C, K), f32),
            jax.ShapeDtypeStruct((C, K), f32),
        ],
        scratch_shapes=[
            pltpu.VMEM((B, C), f32),
            pltpu.VMEM((B, C), f32),
        ],
    )(ct, xt, gamma, beta)

    sc8, sh8, oh = _sc_middle(mc_t, m2, c2, s1t, s2t, gt, bt, float(HW))

    outT = pl.pallas_call(
        _stage_c_body,
        grid=(grid,),
        in_specs=[
            pl.BlockSpec((TH, W, B, C), lambda i: (i, 0, 0, 0)),
            pl.BlockSpec((C, K), lambda i: (0, 0)),
            pl.BlockSpec((C, K), lambda i: (0, 0)),
            pl.BlockSpec((B, K), lambda i: (0, 0)),
        ],
        out_specs=pl.BlockSpec((TH, W, B, C), lambda i: (i, 0, 0, 0)),
        out_shape=jax.ShapeDtypeStruct((H, W, B, C), f32),
    )(xt, sc8, sh8, oh)

    return jnp.transpose(outT, (2, 3, 0, 1))


# speedup vs baseline: 1.6112x; 1.0550x over previous
import functools

import jax
import jax.numpy as jnp
from jax import lax
from jax.experimental import pallas as pl
from jax.experimental.pallas import tpu as pltpu
from jax.experimental.pallas import tpu_sc as plsc

K = 8
EPS = 1e-5
TH = 7
LANES = 16


def _stage_a_body(ct_ref, xt_ref, g_ref, b_ref,
                  mc_ref, s1t_ref, s2t_ref, m2_ref, c2_ref, gt_ref, bt_ref,
                  s1acc, s2acc):
    i = pl.program_id(0)
    n = pl.num_programs(0)
    P = TH * 14
    B, C = s1acc.shape
    ctb = ct_ref[...]
    xb = xt_ref[...].reshape(P, B, C)
    pmc = lax.dot_general(ctb, xb, (((2,), (2,)), ((0,), (0,))),
                          preferred_element_type=jnp.float32)
    pmc = jnp.sum(pmc, axis=0)
    ps1 = jnp.sum(xb, axis=0)
    ps2 = jnp.sum(xb * xb, axis=0)

    pc2 = jnp.sum(jnp.sum(ctb * ctb, axis=0), axis=1, keepdims=True)

    @pl.when(i == 0)
    def _():
        mc_ref[...] = pmc
        s1acc[...] = ps1
        s2acc[...] = ps2
        c2_ref[...] = pc2

    @pl.when(i > 0)
    def _():
        mc_ref[...] += pmc
        s1acc[...] += ps1
        s2acc[...] += ps2
        c2_ref[...] += pc2

    @pl.when(i == n - 1)
    def _():
        s2f = s2acc[...]
        s1t_ref[...] = s1acc[...].T
        s2t_ref[...] = s2f.T
        m2_ref[...] = jnp.sum(s2f, axis=1, keepdims=True)
        gt_ref[...] = g_ref[...].T
        bt_ref[...] = b_ref[...].T


def _stage_c_body(xt_ref, sc8_ref, sh8_ref, oh_ref, o_ref):
    xb = xt_ref[...]
    oh = oh_ref[...]
    dn = (((1,), (1,)), ((), ()))
    sb = lax.dot_general(oh, sc8_ref[...], dn,
                         precision=lax.Precision.HIGHEST,
                         preferred_element_type=jnp.float32)
    tb = lax.dot_general(oh, sh8_ref[...], dn,
                         precision=lax.Precision.HIGHEST,
                         preferred_element_type=jnp.float32)
    o_ref[...] = xb * sb[None, None, :, :] + tb[None, None, :, :]


def _fast_rsqrt(v):
    bits = plsc.bitcast(v, jnp.int32)
    y = plsc.bitcast(jnp.int32(0x5F3759DF) - lax.shift_right_logical(bits, 1),
                     jnp.float32)
    for _ in range(3):
        y = y * (1.5 - 0.5 * v * y * y)
    return y


def _sc_middle(mc_t, m2, c2, s1t, s2t, gt, bt, hw):
    C, B = s1t.shape
    nsub = C // LANES
    nb2 = B // 2
    f32 = jnp.float32
    mesh = plsc.VectorSubcoreMesh(core_axis_name="c", subcore_axis_name="s")

    @functools.partial(
        pl.kernel, mesh=mesh,
        compiler_params=pltpu.CompilerParams(needs_layout_passes=False),
        out_type=[jax.ShapeDtypeStruct((C, K), f32),
                  jax.ShapeDtypeStruct((C, K), f32),
                  jax.ShapeDtypeStruct((B, K), f32)],
        scratch_types=[
            pltpu.VMEM((K, B), f32),
            pltpu.VMEM((B, 1), f32),
            pltpu.VMEM((K, 1), f32),
            pltpu.VMEM((LANES, B), f32),
            pltpu.VMEM((LANES, B), f32),
            pltpu.VMEM((LANES, K), f32),
            pltpu.VMEM((LANES, K), f32),
            pltpu.VMEM((K, LANES), f32),
            pltpu.VMEM((K, LANES), f32),
            pltpu.VMEM((K, LANES), f32),
            pltpu.VMEM((LANES, K), f32),
            pltpu.VMEM((LANES, K), f32),
            pltpu.VMEM((B, K), f32),
            pltpu.SemaphoreType.DMA,
        ],
    )
    def middle(mc_ref, m2_ref, c2_ref, s1_ref, s2_ref, g_ref, b_ref,
               sc8_ref, sh8_ref, oh_ref,
               mc_v, m2_v, c2_v, s1_v, s2_v, g_v, b_v,
               acc1, acc2, cntv, oc_v, os_v, oh_v, sem):
        wid = lax.axis_index("s") * 2 + lax.axis_index("c")

        @pl.when(wid < nsub)
        def _():
            c0 = wid * LANES
            copies = [
                pltpu.async_copy(mc_ref, mc_v, sem),
                pltpu.async_copy(m2_ref, m2_v, sem),
                pltpu.async_copy(c2_ref, c2_v, sem),
                pltpu.async_copy(s1_ref.at[pl.ds(c0, LANES), :], s1_v, sem),
                pltpu.async_copy(s2_ref.at[pl.ds(c0, LANES), :], s2_v, sem),
                pltpu.async_copy(g_ref.at[pl.ds(c0, LANES), :], g_v, sem),
                pltpu.async_copy(b_ref.at[pl.ds(c0, LANES), :], b_v, sem),
            ]
            for cp in copies:
                cp.wait()

            iota = lax.iota(jnp.int32, LANES)
            zeros_i = jnp.zeros((LANES,), jnp.int32)
            rowsel = jnp.bitwise_and(iota, 7)
            half_lo = iota < 8
            big = f32(3.0e38)
            c2v = plsc.load_gather(c2_v, [rowsel, zeros_i])

            for k in range(K):
                acc1[k] = jnp.zeros((LANES,), f32)
                acc2[k] = jnp.zeros((LANES,), f32)
                cntv[k] = jnp.zeros((LANES,), f32)

            def _assign(r):
                b0 = 2 * r
                b1 = b0 + 1
                colidx = jnp.where(half_lo, b0, b1)
                rows = plsc.load_gather(mc_v, [rowsel, colidx])
                m2v = plsc.load_gather(m2_v, [colidx, zeros_i])
                diff = jnp.abs(m2v + c2v - 2.0 * rows)
                dA = jnp.where(half_lo, diff, big)
                kA = plsc.all_reduce_ffs(dA == jnp.min(dA))
                dB = jnp.where(half_lo, big, diff)
                kB = plsc.all_reduce_ffs(dB == jnp.min(dB)) - 8
                return kA, kB

            one = jnp.ones((LANES,), f32)
            zero = jnp.zeros((LANES,), f32)
            oh_mask = (zeros_i + wid) == 0

            def loop1(r, carry):
                b0 = 2 * r
                b1 = b0 + 1
                kA, kB = _assign(r)
                i0 = zeros_i + b0
                i1 = zeros_i + b1
                r1a = plsc.load_gather(s1_v, [iota, i0])
                r1b = plsc.load_gather(s1_v, [iota, i1])
                r2a = plsc.load_gather(s2_v, [iota, i0])
                r2b = plsc.load_gather(s2_v, [iota, i1])
                for k in range(K):
                    wa = jnp.where(kA == k, one, zero)
                    wb = jnp.where(kB == k, one, zero)
                    acc1[k] = acc1[k] + r1a * wa + r1b * wb
                    acc2[k] = acc2[k] + r2a * wa + r2b * wb
                    cntv[k] = cntv[k] + wa + wb
                rowi = jnp.where(half_lo, b0, b1)
                kpair = jnp.where(half_lo, kA + zeros_i, kB + zeros_i)
                ohval = jnp.where(rowsel == kpair, one, zero)
                plsc.store_scatter(oh_v, [rowi, rowsel], ohval, mask=oh_mask)
                return carry

            lax.fori_loop(0, nb2, loop1, 0)

            for k in range(K):
                denom = jnp.maximum(cntv[k] * hw, 1.0)
                mean = acc1[k] / denom
                var = acc2[k] / denom - mean * mean
                inv = _fast_rsqrt(var + EPS)
                gk = plsc.load_gather(g_v, [iota, zeros_i + k])
                bk = plsc.load_gather(b_v, [iota, zeros_i + k])
                sck = gk * inv
                shk = bk - mean * sck
                ki = zeros_i + k
                plsc.store_scatter(oc_v, [iota, ki], sck)
                plsc.store_scatter(os_v, [iota, ki], shk)

            pltpu.sync_copy(oc_v, sc8_ref.at[pl.ds(c0, LANES), :])
            pltpu.sync_copy(os_v, sh8_ref.at[pl.ds(c0, LANES), :])

            @pl.when(wid == 0)
            def _():
                pltpu.sync_copy(oh_v, oh_ref)

    return middle(mc_t, m2, c2, s1t, s2t, gt, bt)


def kernel(x, c, gamma, beta):
    B, C, H, W = x.shape
    HW = H * W
    xt = jnp.transpose(x, (2, 3, 0, 1))
    ct = jnp.transpose(c.reshape(K, C, HW), (2, 0, 1))
    grid = H // TH
    f32 = jnp.float32

    mc_t, s1t, s2t, m2, c2, gt, bt = pl.pallas_call(
        _stage_a_body,
        grid=(grid,),
        in_specs=[
            pl.BlockSpec((TH * W, K, C), lambda i: (i, 0, 0)),
            pl.BlockSpec((TH, W, B, C), lambda i: (i, 0, 0, 0)),
            pl.BlockSpec((K, C), lambda i: (0, 0)),
            pl.BlockSpec((K, C), lambda i: (0, 0)),
        ],
        out_specs=[
            pl.BlockSpec((K, B), lambda i: (0, 0)),
            pl.BlockSpec((C, B), lambda i: (0, 0)),
            pl.BlockSpec((C, B), lambda i: (0, 0)),
            pl.BlockSpec((B, 1), lambda i: (0, 0)),
            pl.BlockSpec((K, 1), lambda i: (0, 0)),
            pl.BlockSpec((C, K), lambda i: (0, 0)),
            pl.BlockSpec((C, K), lambda i: (0, 0)),
        ],
        out_shape=[
            jax.ShapeDtypeStruct((K, B), f32),
            jax.ShapeDtypeStruct((C, B), f32),
            jax.ShapeDtypeStruct((C, B), f32),
            jax.ShapeDtypeStruct((B, 1), f32),
            jax.ShapeDtypeStruct((K, 1), f32),
            jax.ShapeDtypeStruct((C, K), f32),
            jax.ShapeDtypeStruct((C, K), f32),
        ],
        scratch_shapes=[
            pltpu.VMEM((B, C), f32),
            pltpu.VMEM((B, C), f32),
        ],
    )(ct, xt, gamma, beta)

    sc8, sh8, oh = _sc_middle(mc_t, m2, c2, s1t, s2t, gt, bt, float(HW))

    outT = pl.pallas_call(
        _stage_c_body,
        grid=(grid,),
        in_specs=[
            pl.BlockSpec((TH, W, B, C), lambda i: (i, 0, 0, 0)),
            pl.BlockSpec((C, K), lambda i: (0, 0)),
            pl.BlockSpec((C, K), lambda i: (0, 0)),
            pl.BlockSpec((B, K), lambda i: (0, 0)),
        ],
        out_specs=pl.BlockSpec((TH, W, B, C), lambda i: (i, 0, 0, 0)),
        out_shape=jax.ShapeDtypeStruct((H, W, B, C), f32),
    )(xt, sc8, sh8, oh)

    return jnp.transpose(outT, (2, 3, 0, 1))
